# baseline (speedup 1.0000x reference)
import functools

import jax
import jax.numpy as jnp
from jax import lax
from jax.experimental import pallas as pl
from jax.experimental.pallas import tpu as pltpu
from jax.experimental.pallas import tpu_sc as plsc

_NUM_CORES = 2
_NUM_SUBCORES = 16
_NUM_WORKERS = _NUM_CORES * _NUM_SUBCORES

_CH = 128
_GRP = 1024
_NBUF = 2


@functools.lru_cache(maxsize=None)
def _make_lookup(num_emb, dim, batch):
    b_per_w = batch // _NUM_WORKERS
    n_grp = b_per_w // _GRP
    dmas_per_grp = _GRP // _CH
    assert n_grp % _NBUF == 0
    mesh = plsc.VectorSubcoreMesh(core_axis_name="c", subcore_axis_name="s")

    @functools.partial(
        pl.kernel,
        out_type=jax.ShapeDtypeStruct((batch, dim), jnp.float32),
        mesh=mesh,
        scratch_types=[
            pltpu.VMEM((b_per_w,), jnp.int32),
            [pltpu.VMEM((_GRP, dim), jnp.float32) for _ in range(_NBUF)],
            [pltpu.SemaphoreType.DMA for _ in range(_NBUF)],
            [pltpu.SemaphoreType.DMA for _ in range(_NBUF)],
        ],
        compiler_params=pltpu.CompilerParams(use_tc_tiling_on_sc=False),
    )
    def lookup(ids_hbm, table_hbm, out_hbm, idx_v, rows, g_sems, o_sems):
        wid = lax.axis_index("s") * _NUM_CORES + lax.axis_index("c")
        base = wid * b_per_w
        pltpu.sync_copy(ids_hbm.at[pl.ds(base, b_per_w)], idx_v)

        def fire_gathers(g, b):
            for j in range(dmas_per_grp):
                pltpu.async_copy(
                    table_hbm.at[idx_v.at[pl.ds(g * _GRP + j * _CH, _CH)]],
                    rows[b].at[pl.ds(j * _CH, _CH)],
                    g_sems[b],
                )

        def drain_gathers(b):
            for j in range(dmas_per_grp):
                pltpu.make_async_copy(
                    table_hbm.at[idx_v.at[pl.ds(j * _CH, _CH)]],
                    rows[b].at[pl.ds(j * _CH, _CH)],
                    g_sems[b],
                ).wait()

        fire_gathers(0, 0)

        def grp_body(g0, carry):
            for b in range(_NBUF):
                g = g0 + b

                @pl.when(g + 1 < n_grp)
                def _():
                    nb = (b + 1) % _NBUF

                    @pl.when(g + 1 >= _NBUF)
                    def _():
                        pltpu.make_async_copy(
                            rows[nb],
                            out_hbm.at[pl.ds(base, _GRP)],
                            o_sems[nb],
                        ).wait()

                    fire_gathers(g + 1, nb)

                drain_gathers(b)
                pltpu.async_copy(
                    rows[b], out_hbm.at[pl.ds(base + g * _GRP, _GRP)], o_sems[b]
                )
            return carry

        lax.fori_loop(0, n_grp // _NBUF, lambda i, c: grp_body(i * _NBUF, c), 0)
        for b in range(_NBUF):
            pltpu.make_async_copy(
                rows[b], out_hbm.at[pl.ds(base, _GRP)], o_sems[b]
            ).wait()

    return lookup


def kernel(token_ids, weight):
    b0, b1 = token_ids.shape
    num_emb, dim = weight.shape
    batch = b0 * b1
    flat_ids = token_ids.reshape(batch).astype(jnp.int32)
    out = _make_lookup(num_emb, dim, batch)(flat_ids, weight)
    return out.reshape(b0, b1, dim)


# speedup vs baseline: 1.0015x; 1.0015x over previous
import functools

import jax
import jax.numpy as jnp
from jax import lax
from jax.experimental import pallas as pl
from jax.experimental.pallas import tpu as pltpu
from jax.experimental.pallas import tpu_sc as plsc

_NUM_CORES = 2
_NUM_SUBCORES = 16
_NUM_WORKERS = _NUM_CORES * _NUM_SUBCORES

_CH = 1024
_GRP = 1024
_NBUF = 2


@functools.lru_cache(maxsize=None)
def _make_lookup(num_emb, dim, batch):
    b_per_w = batch // _NUM_WORKERS
    n_grp = b_per_w // _GRP
    dmas_per_grp = _GRP // _CH
    assert n_grp % _NBUF == 0
    mesh = plsc.VectorSubcoreMesh(core_axis_name="c", subcore_axis_name="s")

    @functools.partial(
        pl.kernel,
        out_type=jax.ShapeDtypeStruct((batch, dim), jnp.float32),
        mesh=mesh,
        scratch_types=[
            pltpu.VMEM((b_per_w,), jnp.int32),
            [pltpu.VMEM((_GRP, dim), jnp.float32) for _ in range(_NBUF)],
            [pltpu.SemaphoreType.DMA for _ in range(_NBUF)],
            [pltpu.SemaphoreType.DMA for _ in range(_NBUF)],
        ],
        compiler_params=pltpu.CompilerParams(use_tc_tiling_on_sc=False),
    )
    def lookup(ids_hbm, table_hbm, out_hbm, idx_v, rows, g_sems, o_sems):
        wid = lax.axis_index("s") * _NUM_CORES + lax.axis_index("c")
        base = wid * b_per_w
        pltpu.sync_copy(ids_hbm.at[pl.ds(base, b_per_w)], idx_v)

        def fire_gathers(g, b):
            for j in range(dmas_per_grp):
                pltpu.async_copy(
                    table_hbm.at[idx_v.at[pl.ds(g * _GRP + j * _CH, _CH)]],
                    rows[b].at[pl.ds(j * _CH, _CH)],
                    g_sems[b],
                )

        def drain_gathers(b):
            for j in range(dmas_per_grp):
                pltpu.make_async_copy(
                    table_hbm.at[idx_v.at[pl.ds(j * _CH, _CH)]],
                    rows[b].at[pl.ds(j * _CH, _CH)],
                    g_sems[b],
                ).wait()

        fire_gathers(0, 0)

        def grp_body(g0, carry):
            for b in range(_NBUF):
                g = g0 + b

                @pl.when(g + 1 < n_grp)
                def _():
                    nb = (b + 1) % _NBUF

                    @pl.when(g + 1 >= _NBUF)
                    def _():
                        pltpu.make_async_copy(
                            rows[nb],
                            out_hbm.at[pl.ds(base, _GRP)],
                            o_sems[nb],
                        ).wait()

                    fire_gathers(g + 1, nb)

                drain_gathers(b)
                pltpu.async_copy(
                    rows[b], out_hbm.at[pl.ds(base + g * _GRP, _GRP)], o_sems[b]
                )
            return carry

        lax.fori_loop(0, n_grp // _NBUF, lambda i, c: grp_body(i * _NBUF, c), 0)
        for b in range(_NBUF):
            pltpu.make_async_copy(
                rows[b], out_hbm.at[pl.ds(base, _GRP)], o_sems[b]
            ).wait()

    return lookup


def kernel(token_ids, weight):
    b0, b1 = token_ids.shape
    num_emb, dim = weight.shape
    batch = b0 * b1
    flat_ids = token_ids.reshape(batch).astype(jnp.int32)
    out = _make_lookup(num_emb, dim, batch)(flat_ids, weight)
    return out.reshape(b0, b1, dim)
